# baseline (speedup 1.0000x reference)
import jax
import jax.numpy as jnp
from jax import lax
from jax.experimental import pallas as pl
from jax.experimental.pallas import tpu as pltpu
from jax.experimental.pallas import tpu_sc as plsc

_INFO = plsc.get_sparse_core_info()
_NC = _INFO.num_cores
_NS = _INFO.num_subcores
_LANES = _INFO.num_lanes
_NW = _NC * _NS

_NBUF = 4
_BTILE = 128


def _make_body(B, T, D):
    n_bt = B // _BTILE
    d_tiles = D // 8

    def _body(ids_hbm, tab_hbm, pos_hbm, out_hbm, idx_v, rows_v, tbuf, pos_v, gsem, wsem):
        wid = lax.axis_index("s") * _NC + lax.axis_index("c")
        b0 = wid * _BTILE

        pltpu.sync_copy(pos_hbm, pos_v)

        lanes = lax.iota(jnp.int32, _LANES)
        row_base = [lanes + bc * _LANES for bc in range(_BTILE // _LANES)]

        def start_gather(t, slot):
            pltpu.sync_copy(ids_hbm.at[t, pl.ds(b0, _BTILE)], idx_v.at[slot])
            pltpu.make_async_copy(
                tab_hbm.at[idx_v.at[slot]], rows_v.at[slot], gsem.at[slot]
            ).start()

        def wait_gather(slot):
            pltpu.make_async_copy(
                tab_hbm.at[idx_v.at[slot]], rows_v.at[slot], gsem.at[slot]
            ).wait()

        def _write_copy(t, slot):
            return pltpu.make_async_copy(
                tbuf.at[slot],
                out_hbm.at[t, slice(None), pl.ds(wid, 1)],
                wsem.at[slot],
            )

        for b in range(2):
            start_gather(b, b)

        def group_body(g, carry):
            for b in range(_NBUF):
                slot = b
                nslot = (b + 2) % _NBUF
                t = g * _NBUF + b
                tp = t + 2

                @pl.when(tp < T)
                def _():
                    @pl.when(tp >= _NBUF)
                    def _():
                        _write_copy(tp - _NBUF, nslot).wait()

                    start_gather(tp, nslot)

                wait_gather(slot)

                tvec = jnp.full((_LANES,), t, jnp.int32)

                def d_body(d, c2):
                    dvec = jnp.full((_LANES,), d, jnp.int32)
                    pb = plsc.load_gather(pos_v, [tvec, dvec])
                    dt = d // 8
                    di = d % 8
                    for bc in range(_BTILE // _LANES):
                        v = plsc.load_gather(rows_v.at[slot], [row_base[bc], dvec])
                        tbuf[slot, dt, 0, di, pl.ds(bc * _LANES, _LANES)] = v + pb
                    return c2

                lax.fori_loop(0, D, d_body, 0)
                _write_copy(t, slot).start()
            return carry

        lax.fori_loop(0, T // _NBUF, group_body, 0)

        for b in range(_NBUF):
            _write_copy(T - _NBUF + b, b).wait()

    return _body, n_bt, d_tiles


def kernel(token_ids, token_embed, pos_embed):
    B, T = token_ids.shape
    V, D = token_embed.shape
    assert B % (_NW * _BTILE) == 0 or B == _NW * _BTILE
    assert T % _NBUF == 0 and D % 8 == 0

    ids_t = token_ids.T.astype(jnp.int32)

    body, n_bt, d_tiles = _make_body(B, T, D)
    out5 = pl.kernel(
        body,
        out_type=jax.ShapeDtypeStruct((T, d_tiles, n_bt, 8, _BTILE), jnp.float32),
        mesh=plsc.VectorSubcoreMesh(core_axis_name="c", subcore_axis_name="s"),
        scratch_types=[
            pltpu.VMEM((_NBUF, _BTILE), jnp.int32),
            pltpu.VMEM((_NBUF, _BTILE, D), jnp.float32),
            pltpu.VMEM((_NBUF, d_tiles, 1, 8, _BTILE), jnp.float32),
            pltpu.VMEM((T, D), jnp.float32),
            pltpu.SemaphoreType.DMA((_NBUF,)),
            pltpu.SemaphoreType.DMA((_NBUF,)),
        ],
        compiler_params=pltpu.CompilerParams(
            use_tc_tiling_on_sc=False, needs_layout_passes=False
        ),
    )(ids_t, token_embed, pos_embed)
    return out5.transpose(2, 4, 0, 1, 3).reshape(B, T, D)


# speedup vs baseline: 1.4462x; 1.4462x over previous
import jax
import jax.numpy as jnp
from jax import lax
from jax.experimental import pallas as pl
from jax.experimental.pallas import tpu as pltpu
from jax.experimental.pallas import tpu_sc as plsc

_INFO = plsc.get_sparse_core_info()
_NC = _INFO.num_cores
_NS = _INFO.num_subcores
_LANES = _INFO.num_lanes
_NW = _NC * _NS

_NBUF = 4
_BTILE = 128


def _make_body(B, T, D):
    n_bt = B // _BTILE
    d_tiles = D // 8
    n_dv = D // _LANES

    def _body(ids_hbm, tab_hbm, pos_hbm, out_hbm, idx_v, rows_v, tbuf, pos_v, gsem, wsem):
        wid = lax.axis_index("s") * _NC + lax.axis_index("c")
        b0 = wid * _BTILE

        pltpu.sync_copy(pos_hbm, pos_v)

        lanes = lax.iota(jnp.int32, _LANES)
        dvecs = [lanes + j * _LANES for j in range(n_dv)]

        def start_gather(t, slot):
            pltpu.sync_copy(ids_hbm.at[t, pl.ds(b0, _BTILE)], idx_v.at[slot])
            pltpu.make_async_copy(
                tab_hbm.at[idx_v.at[slot]], rows_v.at[slot], gsem.at[slot]
            ).start()

        def wait_gather(slot):
            pltpu.make_async_copy(
                tab_hbm.at[idx_v.at[slot]], rows_v.at[slot], gsem.at[slot]
            ).wait()

        def _write_copies(t, slot):
            return [
                pltpu.make_async_copy(
                    tbuf.at[slot, pl.ds(dt * 8, 8)],
                    out_hbm.at[t, dt, wid],
                    wsem.at[slot],
                )
                for dt in range(d_tiles)
            ]

        for b in range(2):
            start_gather(b, b)

        def group_body(g, carry):
            for b in range(_NBUF):
                slot = b
                nslot = (b + 2) % _NBUF
                t = g * _NBUF + b
                tp = t + 2

                @pl.when(tp < T)
                def _():
                    @pl.when(tp >= _NBUF)
                    def _():
                        for c in _write_copies(tp - _NBUF, nslot):
                            c.wait()

                    start_gather(tp, nslot)

                wait_gather(slot)

                pj = [pos_v[t, pl.ds(j * _LANES, _LANES)] for j in range(n_dv)]

                @plsc.parallel_loop(0, _BTILE, 1, unroll=4)
                def _(r):
                    bivec = jnp.full((_LANES,), r, jnp.int32)
                    for j in range(n_dv):
                        v = rows_v[slot, r, pl.ds(j * _LANES, _LANES)]
                        plsc.store_scatter(
                            tbuf.at[slot], [dvecs[j], bivec], v + pj[j]
                        )

                for c in _write_copies(t, slot):
                    c.start()
            return carry

        lax.fori_loop(0, T // _NBUF, group_body, 0)

        for b in range(_NBUF):
            for c in _write_copies(T - _NBUF + b, b):
                c.wait()

    return _body, n_bt, d_tiles


def kernel(token_ids, token_embed, pos_embed):
    B, T = token_ids.shape
    V, D = token_embed.shape
    assert B == _NW * _BTILE
    assert T % _NBUF == 0 and D % _LANES == 0

    ids_t = token_ids.T.astype(jnp.int32)

    body, n_bt, d_tiles = _make_body(B, T, D)
    out5 = pl.kernel(
        body,
        out_type=jax.ShapeDtypeStruct((T, d_tiles, n_bt, 8, _BTILE), jnp.float32),
        mesh=plsc.VectorSubcoreMesh(core_axis_name="c", subcore_axis_name="s"),
        scratch_types=[
            pltpu.VMEM((_NBUF, _BTILE), jnp.int32),
            pltpu.VMEM((_NBUF, _BTILE, D), jnp.float32),
            pltpu.VMEM((_NBUF, D, _BTILE), jnp.float32),
            pltpu.VMEM((T, D), jnp.float32),
            pltpu.SemaphoreType.DMA((_NBUF,)),
            pltpu.SemaphoreType.DMA((_NBUF,)),
        ],
        compiler_params=pltpu.CompilerParams(
            use_tc_tiling_on_sc=False, needs_layout_passes=False
        ),
    )(ids_t, token_embed, pos_embed)
    return out5.transpose(2, 4, 0, 1, 3).reshape(B, T, D)


# speedup vs baseline: 1.6370x; 1.1320x over previous
import jax
import jax.numpy as jnp
from jax import lax
from jax.experimental import pallas as pl
from jax.experimental.pallas import tpu as pltpu
from jax.experimental.pallas import tpu_sc as plsc

_INFO = plsc.get_sparse_core_info()
_NC = _INFO.num_cores
_NS = _INFO.num_subcores
_LANES = _INFO.num_lanes
_NW = _NC * _NS

_NBUF = 4
_BTILE = 128


def _make_body(B, T, D):
    n_bt = B // _BTILE
    d_tiles = D // 8

    def _body(ids_hbm, tab_hbm, pos_hbm, out_hbm, idx_v, rows_v, tbuf, pos_v, gsem, wsem):
        wid = lax.axis_index("s") * _NC + lax.axis_index("c")
        b0 = wid * _BTILE

        pltpu.sync_copy(pos_hbm, pos_v)

        lanes = lax.iota(jnp.int32, _LANES)
        cks = [(lanes + k) % _LANES for k in range(_LANES)]

        def start_gather(t, slot):
            pltpu.sync_copy(ids_hbm.at[t, pl.ds(b0, _BTILE)], idx_v.at[slot])
            pltpu.make_async_copy(
                tab_hbm.at[idx_v.at[slot]], rows_v.at[slot], gsem.at[slot]
            ).start()

        def wait_gather(slot):
            pltpu.make_async_copy(
                tab_hbm.at[idx_v.at[slot]], rows_v.at[slot], gsem.at[slot]
            ).wait()

        def _write_copies(t, slot):
            return [
                pltpu.make_async_copy(
                    tbuf.at[slot, pl.ds(dt * 8, 8)],
                    out_hbm.at[t, dt, wid],
                    wsem.at[slot],
                )
                for dt in range(d_tiles)
            ]

        for b in range(2):
            start_gather(b, b)

        def group_body(g, carry):
            for b in range(_NBUF):
                slot = b
                nslot = (b + 2) % _NBUF
                t = g * _NBUF + b
                tp = t + 2

                @pl.when(tp < T)
                def _():
                    @pl.when(tp >= _NBUF)
                    def _():
                        for c in _write_copies(tp - _NBUF, nslot):
                            c.wait()

                    start_gather(tp, nslot)

                wait_gather(slot)

                tvec = jnp.full((_LANES,), t, jnp.int32)
                for d0 in range(0, D, _LANES):
                    dvs = [d0 + cks[k] for k in range(_LANES)]
                    pds = [
                        plsc.load_gather(pos_v, [tvec, dvs[k]])
                        for k in range(_LANES)
                    ]

                    @plsc.parallel_loop(0, _BTILE, _LANES, unroll=2)
                    def _(r0):
                        bvec = lanes + r0
                        for k in range(_LANES):
                            v = plsc.load_gather(rows_v.at[slot], [bvec, dvs[k]])
                            plsc.store_scatter(
                                tbuf.at[slot], [dvs[k], bvec], v + pds[k]
                            )

                for c in _write_copies(t, slot):
                    c.start()
            return carry

        lax.fori_loop(0, T // _NBUF, group_body, 0)

        for b in range(_NBUF):
            for c in _write_copies(T - _NBUF + b, b):
                c.wait()

    return _body, n_bt, d_tiles


def kernel(token_ids, token_embed, pos_embed):
    B, T = token_ids.shape
    V, D = token_embed.shape
    assert B == _NW * _BTILE
    assert T % _NBUF == 0 and D % _LANES == 0

    ids_t = token_ids.T.astype(jnp.int32)

    body, n_bt, d_tiles = _make_body(B, T, D)
    out5 = pl.kernel(
        body,
        out_type=jax.ShapeDtypeStruct((T, d_tiles, n_bt, 8, _BTILE), jnp.float32),
        mesh=plsc.VectorSubcoreMesh(core_axis_name="c", subcore_axis_name="s"),
        scratch_types=[
            pltpu.VMEM((_NBUF, _BTILE), jnp.int32),
            pltpu.VMEM((_NBUF, _BTILE, D), jnp.float32),
            pltpu.VMEM((_NBUF, D, _BTILE), jnp.float32),
            pltpu.VMEM((T, D), jnp.float32),
            pltpu.SemaphoreType.DMA((_NBUF,)),
            pltpu.SemaphoreType.DMA((_NBUF,)),
        ],
        compiler_params=pltpu.CompilerParams(
            use_tc_tiling_on_sc=False, needs_layout_passes=False
        ),
    )(ids_t, token_embed, pos_embed)
    return out5.transpose(2, 4, 0, 1, 3).reshape(B, T, D)


# speedup vs baseline: 1.9905x; 1.2159x over previous
import jax
import jax.numpy as jnp
from jax import lax
from jax.experimental import pallas as pl
from jax.experimental.pallas import tpu as pltpu
from jax.experimental.pallas import tpu_sc as plsc

_INFO = plsc.get_sparse_core_info()
_NC = _INFO.num_cores
_NS = _INFO.num_subcores
_LANES = _INFO.num_lanes
_NW = _NC * _NS

_NBUF = 4
_BTILE = 128


def _make_body(B, T, D):
    n_bt = B // _BTILE
    d_tiles = D // 8

    def _body(ids_hbm, tab_hbm, pos_hbm, out_hbm, idx_v, rows_v, tbuf, pos_v, gsem, wsem):
        wid = lax.axis_index("s") * _NC + lax.axis_index("c")
        b0 = wid * _BTILE

        pltpu.sync_copy(pos_hbm, pos_v)

        lanes = lax.iota(jnp.int32, _LANES)
        cks = [(lanes + k) % _LANES for k in range(_LANES)]

        def start_gather(t, slot):
            pltpu.sync_copy(ids_hbm.at[t, pl.ds(b0, _BTILE)], idx_v.at[slot])
            pltpu.make_async_copy(
                tab_hbm.at[idx_v.at[slot]], rows_v.at[slot], gsem.at[slot]
            ).start()

        def wait_gather(slot):
            pltpu.make_async_copy(
                tab_hbm.at[idx_v.at[slot]], rows_v.at[slot], gsem.at[slot]
            ).wait()

        def _write_copies(t, slot):
            return [
                pltpu.make_async_copy(
                    tbuf.at[slot, pl.ds(dt * 8, 8)],
                    out_hbm.at[t, dt, wid],
                    wsem.at[slot],
                )
                for dt in range(d_tiles)
            ]

        for b in range(2):
            start_gather(b, b)

        def group_body(g, carry):
            for b in range(_NBUF):
                slot = b
                nslot = (b + 2) % _NBUF
                t = g * _NBUF + b
                tp = t + 2

                @pl.when(tp < T)
                def _():
                    @pl.when(tp >= _NBUF)
                    def _():
                        for c in _write_copies(tp - _NBUF, nslot):
                            c.wait()

                    start_gather(tp, nslot)

                wait_gather(slot)

                tvec = jnp.full((_LANES,), t, jnp.int32)
                for d0 in range(0, D, _LANES):
                    dvs = [d0 + cks[k] for k in range(_LANES)]
                    pds = [
                        plsc.load_gather(pos_v, [tvec, dvs[k]])
                        for k in range(_LANES)
                    ]

                    @plsc.parallel_loop(0, _BTILE, _LANES, unroll=4)
                    def _(r0):
                        bvec = lanes + r0
                        for k in range(_LANES):
                            v = plsc.load_gather(rows_v.at[slot], [bvec, dvs[k]])
                            plsc.store_scatter(
                                tbuf.at[slot], [dvs[k], bvec], v + pds[k]
                            )

                for c in _write_copies(t, slot):
                    c.start()
            return carry

        lax.fori_loop(0, T // _NBUF, group_body, 0)

        for b in range(_NBUF):
            for c in _write_copies(T - _NBUF + b, b):
                c.wait()

    return _body, n_bt, d_tiles


def kernel(token_ids, token_embed, pos_embed):
    B, T = token_ids.shape
    V, D = token_embed.shape
    assert B == _NW * _BTILE
    assert T % _NBUF == 0 and D % _LANES == 0

    ids_t = token_ids.T.astype(jnp.int32)

    body, n_bt, d_tiles = _make_body(B, T, D)
    out5 = pl.kernel(
        body,
        out_type=jax.ShapeDtypeStruct((T, d_tiles, n_bt, 8, _BTILE), jnp.float32),
        mesh=plsc.VectorSubcoreMesh(core_axis_name="c", subcore_axis_name="s"),
        scratch_types=[
            pltpu.VMEM((_NBUF, _BTILE), jnp.int32),
            pltpu.VMEM((_NBUF, _BTILE, D), jnp.float32),
            pltpu.VMEM((_NBUF, D, _BTILE), jnp.float32),
            pltpu.VMEM((T, D), jnp.float32),
            pltpu.SemaphoreType.DMA((_NBUF,)),
            pltpu.SemaphoreType.DMA((_NBUF,)),
        ],
        compiler_params=pltpu.CompilerParams(
            use_tc_tiling_on_sc=False, needs_layout_passes=False
        ),
    )(ids_t, token_embed, pos_embed)
    return out5.transpose(2, 4, 0, 1, 3).reshape(B, T, D)


# speedup vs baseline: 2.2812x; 1.1460x over previous
import jax
import jax.numpy as jnp
from jax import lax
from jax.experimental import pallas as pl
from jax.experimental.pallas import tpu as pltpu
from jax.experimental.pallas import tpu_sc as plsc

_INFO = plsc.get_sparse_core_info()
_NC = _INFO.num_cores
_NS = _INFO.num_subcores
_LANES = _INFO.num_lanes
_NW = _NC * _NS

_NBUF = 4
_BTILE = 128


def _make_body(B, T, D):
    n_bt = B // _BTILE
    d_tiles = D // 8

    def _body(ids_hbm, tab_hbm, pos_hbm, out_hbm, idx_v, rows_v, tbuf, pos_v, gsem, wsem):
        wid = lax.axis_index("s") * _NC + lax.axis_index("c")
        b0 = wid * _BTILE

        pltpu.sync_copy(pos_hbm, pos_v)

        lanes = lax.iota(jnp.int32, _LANES)
        cks = [(lanes + k) % _LANES for k in range(_LANES)]

        pltpu.sync_copy(ids_hbm.at[:, pl.ds(b0, _BTILE)], idx_v)

        def start_gather(t, slot):
            pltpu.make_async_copy(
                tab_hbm.at[idx_v.at[t]], rows_v.at[slot], gsem.at[slot]
            ).start()

        def wait_gather(t, slot):
            pltpu.make_async_copy(
                tab_hbm.at[idx_v.at[t]], rows_v.at[slot], gsem.at[slot]
            ).wait()

        def _write_copies(t, slot):
            return [
                pltpu.make_async_copy(
                    tbuf.at[slot, pl.ds(dt * 8, 8)],
                    out_hbm.at[t, dt, wid],
                    wsem.at[slot],
                )
                for dt in range(d_tiles)
            ]

        for b in range(2):
            start_gather(b, b)

        def group_body(g, carry):
            for b in range(_NBUF):
                slot = b
                nslot = (b + 2) % _NBUF
                t = g * _NBUF + b
                tp = t + 2

                @pl.when(tp < T)
                def _():
                    @pl.when(tp >= _NBUF)
                    def _():
                        for c in _write_copies(tp - _NBUF, nslot):
                            c.wait()

                    start_gather(tp, nslot)

                wait_gather(t, slot)

                tvec = jnp.full((_LANES,), t, jnp.int32)
                for d0 in range(0, D, _LANES):
                    dvs = [d0 + cks[k] for k in range(_LANES)]
                    pds = [
                        plsc.load_gather(pos_v, [tvec, dvs[k]])
                        for k in range(_LANES)
                    ]

                    @plsc.parallel_loop(0, _BTILE, _LANES, unroll=4)
                    def _(r0):
                        bvec = lanes + r0
                        for k in range(_LANES):
                            v = plsc.load_gather(rows_v.at[slot], [bvec, dvs[k]])
                            plsc.store_scatter(
                                tbuf.at[slot], [dvs[k], bvec], v + pds[k]
                            )

                for c in _write_copies(t, slot):
                    c.start()
            return carry

        lax.fori_loop(0, T // _NBUF, group_body, 0)

        for b in range(_NBUF):
            for c in _write_copies(T - _NBUF + b, b):
                c.wait()

    return _body, n_bt, d_tiles


def kernel(token_ids, token_embed, pos_embed):
    B, T = token_ids.shape
    V, D = token_embed.shape
    assert B == _NW * _BTILE
    assert T % _NBUF == 0 and D % _LANES == 0

    ids_t = token_ids.T.astype(jnp.int32)

    body, n_bt, d_tiles = _make_body(B, T, D)
    out5 = pl.kernel(
        body,
        out_type=jax.ShapeDtypeStruct((T, d_tiles, n_bt, 8, _BTILE), jnp.float32),
        mesh=plsc.VectorSubcoreMesh(core_axis_name="c", subcore_axis_name="s"),
        scratch_types=[
            pltpu.VMEM((T, _BTILE), jnp.int32),
            pltpu.VMEM((_NBUF, _BTILE, D), jnp.float32),
            pltpu.VMEM((_NBUF, D, _BTILE), jnp.float32),
            pltpu.VMEM((T, D), jnp.float32),
            pltpu.SemaphoreType.DMA((_NBUF,)),
            pltpu.SemaphoreType.DMA((_NBUF,)),
        ],
        compiler_params=pltpu.CompilerParams(
            use_tc_tiling_on_sc=False, needs_layout_passes=False
        ),
    )(ids_t, token_embed, pos_embed)
    return out5.transpose(2, 4, 0, 1, 3).reshape(B, T, D)
